# baseline (speedup 1.0000x reference)
import dataclasses
import functools

import jax
import jax.numpy as jnp
from jax import lax
from jax.experimental import pallas as pl
from jax.experimental.pallas import tpu as pltpu
from jax.experimental.pallas import tpu_sc as plsc

N = 10000
E = 160000
F_IN = 256
F_HID = 512
F_OUT = 256
NC = 2
NS = 16
LANES = 16
HALF = 128

EB = 128
NBLK = E // EB
KMAX = 80

ROWS = 624
ROWS_LAST = N - ROWS * (NS - 1)
DEGW = 128


def _sc_compiler_params():
  cp = pltpu.CompilerParams()
  if "needs_layout_passes" in pltpu.CompilerParams.__dataclass_fields__:
    cp = dataclasses.replace(cp, needs_layout_passes=False)
  return cp


def _vmesh():
  return plsc.VectorSubcoreMesh(core_axis_name="c", subcore_axis_name="s")


def _zero_fill(ref, rows, width):
  @pl.loop(0, rows)
  def _(i):
    for j in range(width // LANES):
      ref[i, pl.ds(j * LANES, LANES)] = jnp.zeros((LANES,), jnp.float32)


_ZCHUNKS = ((0, 128), (128, 128), (256, 128), (384, 128), (512, 112))


def _zero_spmem(acc_sp, zbuf, s, zsem):
  base = ROWS * s
  for off, n in _ZCHUNKS:
    pltpu.async_copy(zbuf.at[pl.ds(0, n)], acc_sp.at[pl.ds(base + off, n)],
                     zsem)

  @pl.when(s == NS - 1)
  def _():
    pltpu.async_copy(zbuf.at[pl.ds(0, ROWS_LAST - ROWS)],
                     acc_sp.at[pl.ds(ROWS * NS, ROWS_LAST - ROWS)], zsem)

  for off, n in _ZCHUNKS:
    pltpu.make_async_copy(zbuf.at[pl.ds(0, n)],
                          acc_sp.at[pl.ds(base + off, n)], zsem).wait()

  @pl.when(s == NS - 1)
  def _():
    pltpu.make_async_copy(zbuf.at[pl.ds(0, ROWS_LAST - ROWS)],
                          acc_sp.at[pl.ds(ROWS * NS, ROWS_LAST - ROWS)],
                          zsem).wait()


def _copy_out(acc_sp, out_hbm, c, s):
  row0 = ROWS * s
  pltpu.sync_copy(acc_sp.at[pl.ds(row0, ROWS)],
                  out_hbm.at[c].at[pl.ds(row0, ROWS)])

  @pl.when(s == NS - 1)
  def _():
    row1 = ROWS * NS
    pltpu.sync_copy(acc_sp.at[pl.ds(row1, ROWS_LAST - ROWS)],
                    out_hbm.at[c].at[pl.ds(row1, ROWS_LAST - ROWS)])


HROWS = 80


def _deg_body(src_hbm, dst_hbm, cnt_hbm, acc_sp,
              idx0, idx1, hist_v, ident_v, isem0, isem1):
  c = lax.axis_index("c")
  s = lax.axis_index("s")
  idx_vs = (idx0, idx1)
  isems = (isem0, isem1)

  _zero_fill(hist_v, HROWS, DEGW)
  for g in range(HROWS // LANES):
    ident_v[pl.ds(g * LANES, LANES)] = (
        lax.iota(jnp.int32, LANES) + g * LANES)

  @pl.when(s == 0)
  def _():
    pltpu.sync_copy(hist_v, acc_sp)
  plsc.subcore_barrier()

  def valid(k):
    return (k * NS + s) < NBLK

  def issue_idx(k, slot):
    sl = pl.ds((k * NS + s) * EB, EB)

    @pl.when(c == 0)
    def _():
      pltpu.async_copy(src_hbm.at[sl], idx_vs[slot], isems[slot])

    @pl.when(c == 1)
    def _():
      pltpu.async_copy(dst_hbm.at[sl], idx_vs[slot], isems[slot])

  def wait_idx(slot):
    pltpu.make_async_copy(src_hbm.at[pl.ds(0, EB)], idx_vs[slot],
                          isems[slot]).wait()

  issue_idx(0, 0)
  ones16 = jnp.ones((LANES,), jnp.float32)

  @pl.loop(0, KMAX, step=2)
  def _(k0):
    for dk in (0, 1):
      k = k0 + dk
      r, o = dk, 1 - dk

      @pl.when(valid(k + 1))
      def _():
        issue_idx(k + 1, o)

      @pl.when(valid(k))
      def _():
        wait_idx(r)
        for g in range(EB // LANES):
          idx16 = idx_vs[r][pl.ds(g * LANES, LANES)]
          row16 = lax.shift_right_logical(idx16, 7)
          col16 = lax.bitwise_and(idx16, 127)
          plsc.addupdate_scatter(hist_v, [row16, col16], ones16)

  pltpu.sync_copy(hist_v, acc_sp.at[ident_v], add=True)
  plsc.subcore_barrier()

  @pl.when(s == 0)
  def _():
    pltpu.sync_copy(acc_sp, cnt_hbm.at[c])


def _sc_degrees(src, dst):
  kern = pl.kernel(
      _deg_body,
      out_type=jax.ShapeDtypeStruct((NC, HROWS, DEGW), jnp.float32),
      mesh=_vmesh(),
      scratch_types=[
          pltpu.VMEM_SHARED((HROWS, DEGW), jnp.float32),
          pltpu.VMEM((EB,), jnp.int32),
          pltpu.VMEM((EB,), jnp.int32),
          pltpu.VMEM((HROWS, DEGW), jnp.float32),
          pltpu.VMEM((HROWS,), jnp.int32),
          pltpu.SemaphoreType.DMA,
          pltpu.SemaphoreType.DMA,
      ],
      compiler_params=_sc_compiler_params(),
  )
  return kern(src, dst)


def _agg_body(tbl_hbm, src_hbm, dst_hbm, w_hbm, scale_hbm, out_hbm, acc_sp,
              idx0, idx1, dst0, dst1, w0, w1, rows0, rows1, dstS, scale_v,
              isem0, isem1, gsem0, gsem1, ssem):
  c = lax.axis_index("c")
  s = lax.axis_index("s")
  idx_vs = (idx0, idx1)
  dst_vs = (dst0, dst1)
  w_vs = (w0, w1)
  rows_vs = (rows0, rows1)
  isems = (isem0, isem1)
  gsems = (gsem0, gsem1)

  _zero_fill(rows0, EB, HALF)
  _zero_spmem(acc_sp, rows0, s, ssem)
  pltpu.sync_copy(scale_hbm, scale_v)
  plsc.subcore_barrier()

  def valid(k):
    return (k * NS + s) < NBLK

  def issue_idx(k, slot):
    sl = pl.ds((k * NS + s) * EB, EB)
    pltpu.async_copy(src_hbm.at[sl], idx_vs[slot], isems[slot])
    pltpu.async_copy(dst_hbm.at[sl], dst_vs[slot], isems[slot])
    pltpu.async_copy(w_hbm.at[sl], w_vs[slot], isems[slot])

  def wait_idx(slot):
    pltpu.make_async_copy(src_hbm.at[pl.ds(0, EB)], idx_vs[slot],
                          isems[slot]).wait()
    pltpu.make_async_copy(dst_hbm.at[pl.ds(0, EB)], dst_vs[slot],
                          isems[slot]).wait()
    pltpu.make_async_copy(w_hbm.at[pl.ds(0, EB)], w_vs[slot],
                          isems[slot]).wait()

  def transform_idx(slot):
    for g in range(EB // LANES):
      gsl = pl.ds(g * LANES, LANES)
      s16 = idx_vs[slot][gsl]
      w_vs[slot][gsl] = w_vs[slot][gsl] * plsc.load_gather(scale_v, [s16])
      idx_vs[slot][gsl] = s16 + s16 + c

  def issue_gather(slot):
    pltpu.async_copy(tbl_hbm.at[idx_vs[slot]], rows_vs[slot], gsems[slot])

  def wait_gather(slot):
    pltpu.make_async_copy(tbl_hbm.at[idx_vs[slot]], rows_vs[slot],
                          gsems[slot]).wait()

  issue_idx(0, 0)
  wait_idx(0)
  transform_idx(0)
  issue_gather(0)
  issue_idx(1, 1)

  @pl.loop(0, KMAX, step=2)
  def _(k0):
    for dk in (0, 1):
      k = k0 + dk
      r, o = dk, 1 - dk

      @pl.when(jnp.logical_and(k >= 1, valid(k - 1)))
      def _():
        pltpu.make_async_copy(rows_vs[o], acc_sp.at[dstS], ssem).wait()

      @pl.when(valid(k + 1))
      def _():
        wait_idx(o)
        transform_idx(o)
        issue_gather(o)

      @pl.when(valid(k))
      def _():
        wait_gather(r)

        for g in range(EB // LANES):
          gsl = pl.ds(g * LANES, LANES)
          dstS[gsl] = dst_vs[r][gsl]

        @plsc.parallel_loop(0, EB, unroll=4)
        def _(i):
          wspl = plsc.load_gather(w_vs[r], [jnp.broadcast_to(i, (LANES,))])
          for j in range(HALF // LANES):
            jsl = pl.ds(j * LANES, LANES)
            rows_vs[r][i, jsl] = rows_vs[r][i, jsl] * wspl

        pltpu.async_copy(rows_vs[r], acc_sp.at[dstS], ssem, add=True)

      @pl.when(valid(k + 2))
      def _():
        issue_idx(k + 2, r)

  plsc.subcore_barrier()
  _copy_out(acc_sp, out_hbm, c, s)


def _sc_agg(table, src, dst, w, scale):
  kern = pl.kernel(
      _agg_body,
      out_type=jax.ShapeDtypeStruct((NC, N, HALF), jnp.float32),
      mesh=_vmesh(),
      scratch_types=[
          pltpu.VMEM_SHARED((N, HALF), jnp.float32),
          pltpu.VMEM((EB,), jnp.int32),
          pltpu.VMEM((EB,), jnp.int32),
          pltpu.VMEM((EB,), jnp.int32),
          pltpu.VMEM((EB,), jnp.int32),
          pltpu.VMEM((EB,), jnp.float32),
          pltpu.VMEM((EB,), jnp.float32),
          pltpu.VMEM((EB, HALF), jnp.float32),
          pltpu.VMEM((EB, HALF), jnp.float32),
          pltpu.VMEM((EB,), jnp.int32),
          pltpu.VMEM((N,), jnp.float32),
          pltpu.SemaphoreType.DMA,
          pltpu.SemaphoreType.DMA,
          pltpu.SemaphoreType.DMA,
          pltpu.SemaphoreType.DMA,
          pltpu.SemaphoreType.DMA,
      ],
      compiler_params=_sc_compiler_params(),
  )
  return kern(table, src, dst, w, scale)


def _norm_body(cnt_ref, out_ref):
  flat = cnt_ref[...].reshape(NC, HROWS * DEGW)[:, :N]
  out_ref[...] = lax.rsqrt(jnp.maximum(flat, 1.0))[:, :, None]


def _tc_norm(cnt):
  return pl.pallas_call(
      _norm_body,
      out_shape=jax.ShapeDtypeStruct((NC, N, 1), jnp.float32),
  )(cnt)


_MM_BLK = 1000


def _mm_body(agg_ref, ndst_ref, w1_ref, b1_ref, w2_ref, out_ref):
  a = jnp.concatenate([agg_ref[0], agg_ref[1]], axis=-1)
  a = a * ndst_ref[0]
  h = jnp.dot(a, w1_ref[...], preferred_element_type=jnp.float32,
              precision=lax.Precision.DEFAULT)
  h = jnp.maximum(h + b1_ref[...][None, :], 0.0)
  out_ref[...] = jnp.dot(h, w2_ref[...], preferred_element_type=jnp.float32,
                         precision=lax.Precision.DEFAULT)


def _tc_mm(agg, norm3, W1, b1, W2):
  return pl.pallas_call(
      _mm_body,
      grid=(N // _MM_BLK,),
      in_specs=[
          pl.BlockSpec((NC, _MM_BLK, HALF), lambda i: (0, i, 0)),
          pl.BlockSpec((1, _MM_BLK, 1), lambda i: (1, i, 0)),
          pl.BlockSpec((F_IN, F_HID), lambda i: (0, 0)),
          pl.BlockSpec((F_HID,), lambda i: (0,)),
          pl.BlockSpec((F_HID, F_OUT), lambda i: (0, 0)),
      ],
      out_specs=pl.BlockSpec((_MM_BLK, F_OUT), lambda i: (i, 0)),
      out_shape=jax.ShapeDtypeStruct((N, F_OUT), jnp.float32),
  )(agg, norm3, W1, b1, W2)


def _out_body(agg_ref, ndst_ref, b2_ref, out_ref):
  o = jnp.concatenate([agg_ref[0], agg_ref[1]], axis=-1)
  out_ref[...] = o * ndst_ref[0] + b2_ref[...][None, :]


def _tc_out(agg, norm3, b2):
  return pl.pallas_call(
      _out_body,
      grid=(N // _MM_BLK,),
      in_specs=[
          pl.BlockSpec((NC, _MM_BLK, HALF), lambda i: (0, i, 0)),
          pl.BlockSpec((1, _MM_BLK, 1), lambda i: (1, i, 0)),
          pl.BlockSpec((F_OUT,), lambda i: (0,)),
      ],
      out_specs=pl.BlockSpec((_MM_BLK, F_OUT), lambda i: (i, 0)),
      out_shape=jax.ShapeDtypeStruct((N, F_OUT), jnp.float32),
  )(agg, norm3, b2)


def kernel(node_feats, edge_index, edge_weight, W1, b1, W2, b2):
  ei = edge_index.astype(jnp.int32)
  src = ei[0]
  dst = ei[1]
  w = edge_weight.astype(jnp.float32)

  cnt = _sc_degrees(src, dst)
  norm3 = _tc_norm(cnt)
  nsrc = norm3[0, :, 0]

  agg1 = _sc_agg(node_feats.reshape(NC * N, HALF), src, dst, w, nsrc)
  g2 = _tc_mm(agg1, norm3, W1, b1, W2)
  agg2 = _sc_agg(g2.reshape(NC * N, HALF), src, dst, w, nsrc)
  return _tc_out(agg2, norm3, b2)


# speedup vs baseline: 1.0006x; 1.0006x over previous
import dataclasses
import functools

import jax
import jax.numpy as jnp
from jax import lax
from jax.experimental import pallas as pl
from jax.experimental.pallas import tpu as pltpu
from jax.experimental.pallas import tpu_sc as plsc

N = 10000
E = 160000
F_IN = 256
F_HID = 512
F_OUT = 256
NC = 2
NS = 16
LANES = 16
HALF = 128

EB = 128
NBLK = E // EB
KMAX = 80

ROWS = 624
ROWS_LAST = N - ROWS * (NS - 1)
DEGW = 128


def _sc_compiler_params():
  cp = pltpu.CompilerParams()
  if "needs_layout_passes" in pltpu.CompilerParams.__dataclass_fields__:
    cp = dataclasses.replace(cp, needs_layout_passes=False)
  return cp


def _vmesh():
  return plsc.VectorSubcoreMesh(core_axis_name="c", subcore_axis_name="s")


def _zero_fill(ref, rows, width):
  @pl.loop(0, rows)
  def _(i):
    for j in range(width // LANES):
      ref[i, pl.ds(j * LANES, LANES)] = jnp.zeros((LANES,), jnp.float32)


_ZCHUNKS = ((0, 128), (128, 128), (256, 128), (384, 128), (512, 112))


def _zero_spmem(acc_sp, zbuf, s, zsem):
  base = ROWS * s
  for off, n in _ZCHUNKS:
    pltpu.async_copy(zbuf.at[pl.ds(0, n)], acc_sp.at[pl.ds(base + off, n)],
                     zsem)

  @pl.when(s == NS - 1)
  def _():
    pltpu.async_copy(zbuf.at[pl.ds(0, ROWS_LAST - ROWS)],
                     acc_sp.at[pl.ds(ROWS * NS, ROWS_LAST - ROWS)], zsem)

  for off, n in _ZCHUNKS:
    pltpu.make_async_copy(zbuf.at[pl.ds(0, n)],
                          acc_sp.at[pl.ds(base + off, n)], zsem).wait()

  @pl.when(s == NS - 1)
  def _():
    pltpu.make_async_copy(zbuf.at[pl.ds(0, ROWS_LAST - ROWS)],
                          acc_sp.at[pl.ds(ROWS * NS, ROWS_LAST - ROWS)],
                          zsem).wait()


def _copy_out(acc_sp, out_hbm, c, s):
  row0 = ROWS * s
  pltpu.sync_copy(acc_sp.at[pl.ds(row0, ROWS)],
                  out_hbm.at[c].at[pl.ds(row0, ROWS)])

  @pl.when(s == NS - 1)
  def _():
    row1 = ROWS * NS
    pltpu.sync_copy(acc_sp.at[pl.ds(row1, ROWS_LAST - ROWS)],
                    out_hbm.at[c].at[pl.ds(row1, ROWS_LAST - ROWS)])


HROWS = 80


def _deg_body(src_hbm, dst_hbm, cnt_hbm, acc_sp,
              idx0, idx1, hist_v, ident_v, isem0, isem1):
  c = lax.axis_index("c")
  s = lax.axis_index("s")
  idx_vs = (idx0, idx1)
  isems = (isem0, isem1)

  _zero_fill(hist_v, HROWS, DEGW)
  for g in range(HROWS // LANES):
    ident_v[pl.ds(g * LANES, LANES)] = (
        lax.iota(jnp.int32, LANES) + g * LANES)

  @pl.when(s == 0)
  def _():
    pltpu.sync_copy(hist_v, acc_sp)
  plsc.subcore_barrier()

  def valid(k):
    return (k * NS + s) < NBLK

  def issue_idx(k, slot):
    sl = pl.ds((k * NS + s) * EB, EB)

    @pl.when(c == 0)
    def _():
      pltpu.async_copy(src_hbm.at[sl], idx_vs[slot], isems[slot])

    @pl.when(c == 1)
    def _():
      pltpu.async_copy(dst_hbm.at[sl], idx_vs[slot], isems[slot])

  def wait_idx(slot):
    pltpu.make_async_copy(src_hbm.at[pl.ds(0, EB)], idx_vs[slot],
                          isems[slot]).wait()

  issue_idx(0, 0)
  ones16 = jnp.ones((LANES,), jnp.float32)

  @pl.loop(0, KMAX, step=2)
  def _(k0):
    for dk in (0, 1):
      k = k0 + dk
      r, o = dk, 1 - dk

      @pl.when(valid(k + 1))
      def _():
        issue_idx(k + 1, o)

      @pl.when(valid(k))
      def _():
        wait_idx(r)
        for g in range(EB // LANES):
          idx16 = idx_vs[r][pl.ds(g * LANES, LANES)]
          row16 = lax.shift_right_logical(idx16, 7)
          col16 = lax.bitwise_and(idx16, 127)
          plsc.addupdate_scatter(hist_v, [row16, col16], ones16)

  pltpu.sync_copy(hist_v, acc_sp.at[ident_v], add=True)
  plsc.subcore_barrier()

  @pl.when(s == 0)
  def _():
    pltpu.sync_copy(acc_sp, cnt_hbm.at[c])


def _sc_degrees(src, dst):
  kern = pl.kernel(
      _deg_body,
      out_type=jax.ShapeDtypeStruct((NC, HROWS, DEGW), jnp.float32),
      mesh=_vmesh(),
      scratch_types=[
          pltpu.VMEM_SHARED((HROWS, DEGW), jnp.float32),
          pltpu.VMEM((EB,), jnp.int32),
          pltpu.VMEM((EB,), jnp.int32),
          pltpu.VMEM((HROWS, DEGW), jnp.float32),
          pltpu.VMEM((HROWS,), jnp.int32),
          pltpu.SemaphoreType.DMA,
          pltpu.SemaphoreType.DMA,
      ],
      compiler_params=_sc_compiler_params(),
  )
  return kern(src, dst)


def _agg_body(tbl_hbm, src_hbm, dst_hbm, w_hbm, scale_hbm, out_hbm, acc_sp,
              idx0, idx1, dst0, dst1, w0, w1, rows0, rows1, dstS, scale_v,
              isem0, isem1, gsem0, gsem1, ssem):
  c = lax.axis_index("c")
  s = lax.axis_index("s")
  idx_vs = (idx0, idx1)
  dst_vs = (dst0, dst1)
  w_vs = (w0, w1)
  rows_vs = (rows0, rows1)
  isems = (isem0, isem1)
  gsems = (gsem0, gsem1)

  _zero_fill(rows0, EB, HALF)
  _zero_spmem(acc_sp, rows0, s, ssem)
  pltpu.sync_copy(scale_hbm, scale_v)
  plsc.subcore_barrier()

  def valid(k):
    return (k * NS + s) < NBLK

  def issue_idx(k, slot):
    sl = pl.ds((k * NS + s) * EB, EB)
    pltpu.async_copy(src_hbm.at[sl], idx_vs[slot], isems[slot])
    pltpu.async_copy(dst_hbm.at[sl], dst_vs[slot], isems[slot])
    pltpu.async_copy(w_hbm.at[sl], w_vs[slot], isems[slot])

  def wait_idx(slot):
    pltpu.make_async_copy(src_hbm.at[pl.ds(0, EB)], idx_vs[slot],
                          isems[slot]).wait()
    pltpu.make_async_copy(dst_hbm.at[pl.ds(0, EB)], dst_vs[slot],
                          isems[slot]).wait()
    pltpu.make_async_copy(w_hbm.at[pl.ds(0, EB)], w_vs[slot],
                          isems[slot]).wait()

  def transform_idx(slot):
    for g in range(EB // LANES):
      gsl = pl.ds(g * LANES, LANES)
      s16 = idx_vs[slot][gsl]
      idx_vs[slot][gsl] = s16 + s16 + c

  def fold_weight(slot):
    for g in range(EB // LANES):
      gsl = pl.ds(g * LANES, LANES)
      s16 = lax.shift_right_logical(idx_vs[slot][gsl], 1)
      w_vs[slot][gsl] = w_vs[slot][gsl] * plsc.load_gather(scale_v, [s16])

  def issue_gather(slot):
    pltpu.async_copy(tbl_hbm.at[idx_vs[slot]], rows_vs[slot], gsems[slot])

  def wait_gather(slot):
    pltpu.make_async_copy(tbl_hbm.at[idx_vs[slot]], rows_vs[slot],
                          gsems[slot]).wait()

  issue_idx(0, 0)
  wait_idx(0)
  transform_idx(0)
  issue_gather(0)
  fold_weight(0)
  issue_idx(1, 1)

  @pl.loop(0, KMAX, step=2)
  def _(k0):
    for dk in (0, 1):
      k = k0 + dk
      r, o = dk, 1 - dk

      @pl.when(jnp.logical_and(k >= 1, valid(k - 1)))
      def _():
        pltpu.make_async_copy(rows_vs[o], acc_sp.at[dstS], ssem).wait()

      @pl.when(valid(k + 1))
      def _():
        wait_idx(o)
        transform_idx(o)
        issue_gather(o)
        fold_weight(o)

      @pl.when(valid(k))
      def _():
        wait_gather(r)

        for g in range(EB // LANES):
          gsl = pl.ds(g * LANES, LANES)
          dstS[gsl] = dst_vs[r][gsl]

        @plsc.parallel_loop(0, EB, unroll=4)
        def _(i):
          wspl = plsc.load_gather(w_vs[r], [jnp.broadcast_to(i, (LANES,))])
          for j in range(HALF // LANES):
            jsl = pl.ds(j * LANES, LANES)
            rows_vs[r][i, jsl] = rows_vs[r][i, jsl] * wspl

        pltpu.async_copy(rows_vs[r], acc_sp.at[dstS], ssem, add=True)

      @pl.when(valid(k + 2))
      def _():
        issue_idx(k + 2, r)

  plsc.subcore_barrier()
  _copy_out(acc_sp, out_hbm, c, s)


def _sc_agg(table, src, dst, w, scale):
  kern = pl.kernel(
      _agg_body,
      out_type=jax.ShapeDtypeStruct((NC, N, HALF), jnp.float32),
      mesh=_vmesh(),
      scratch_types=[
          pltpu.VMEM_SHARED((N, HALF), jnp.float32),
          pltpu.VMEM((EB,), jnp.int32),
          pltpu.VMEM((EB,), jnp.int32),
          pltpu.VMEM((EB,), jnp.int32),
          pltpu.VMEM((EB,), jnp.int32),
          pltpu.VMEM((EB,), jnp.float32),
          pltpu.VMEM((EB,), jnp.float32),
          pltpu.VMEM((EB, HALF), jnp.float32),
          pltpu.VMEM((EB, HALF), jnp.float32),
          pltpu.VMEM((EB,), jnp.int32),
          pltpu.VMEM((N,), jnp.float32),
          pltpu.SemaphoreType.DMA,
          pltpu.SemaphoreType.DMA,
          pltpu.SemaphoreType.DMA,
          pltpu.SemaphoreType.DMA,
          pltpu.SemaphoreType.DMA,
      ],
      compiler_params=_sc_compiler_params(),
  )
  return kern(table, src, dst, w, scale)


def _norm_body(cnt_ref, out_ref):
  flat = cnt_ref[...].reshape(NC, HROWS * DEGW)[:, :N]
  out_ref[...] = lax.rsqrt(jnp.maximum(flat, 1.0))[:, :, None]


def _tc_norm(cnt):
  return pl.pallas_call(
      _norm_body,
      out_shape=jax.ShapeDtypeStruct((NC, N, 1), jnp.float32),
  )(cnt)


_MM_BLK = 1000


def _mm_body(agg_ref, ndst_ref, w1_ref, b1_ref, w2_ref, out_ref):
  a = jnp.concatenate([agg_ref[0], agg_ref[1]], axis=-1)
  a = a * ndst_ref[0]
  h = jnp.dot(a, w1_ref[...], preferred_element_type=jnp.float32,
              precision=lax.Precision.DEFAULT)
  h = jnp.maximum(h + b1_ref[...][None, :], 0.0)
  out_ref[...] = jnp.dot(h, w2_ref[...], preferred_element_type=jnp.float32,
                         precision=lax.Precision.DEFAULT)


def _tc_mm(agg, norm3, W1, b1, W2):
  return pl.pallas_call(
      _mm_body,
      grid=(N // _MM_BLK,),
      in_specs=[
          pl.BlockSpec((NC, _MM_BLK, HALF), lambda i: (0, i, 0)),
          pl.BlockSpec((1, _MM_BLK, 1), lambda i: (1, i, 0)),
          pl.BlockSpec((F_IN, F_HID), lambda i: (0, 0)),
          pl.BlockSpec((F_HID,), lambda i: (0,)),
          pl.BlockSpec((F_HID, F_OUT), lambda i: (0, 0)),
      ],
      out_specs=pl.BlockSpec((_MM_BLK, F_OUT), lambda i: (i, 0)),
      out_shape=jax.ShapeDtypeStruct((N, F_OUT), jnp.float32),
  )(agg, norm3, W1, b1, W2)


def _out_body(agg_ref, ndst_ref, b2_ref, out_ref):
  o = jnp.concatenate([agg_ref[0], agg_ref[1]], axis=-1)
  out_ref[...] = o * ndst_ref[0] + b2_ref[...][None, :]


def _tc_out(agg, norm3, b2):
  return pl.pallas_call(
      _out_body,
      grid=(N // _MM_BLK,),
      in_specs=[
          pl.BlockSpec((NC, _MM_BLK, HALF), lambda i: (0, i, 0)),
          pl.BlockSpec((1, _MM_BLK, 1), lambda i: (1, i, 0)),
          pl.BlockSpec((F_OUT,), lambda i: (0,)),
      ],
      out_specs=pl.BlockSpec((_MM_BLK, F_OUT), lambda i: (i, 0)),
      out_shape=jax.ShapeDtypeStruct((N, F_OUT), jnp.float32),
  )(agg, norm3, b2)


def kernel(node_feats, edge_index, edge_weight, W1, b1, W2, b2):
  ei = edge_index.astype(jnp.int32)
  src = ei[0]
  dst = ei[1]
  w = edge_weight.astype(jnp.float32)

  cnt = _sc_degrees(src, dst)
  norm3 = _tc_norm(cnt)
  nsrc = norm3[0, :, 0]

  agg1 = _sc_agg(node_feats.reshape(NC * N, HALF), src, dst, w, nsrc)
  g2 = _tc_mm(agg1, norm3, W1, b1, W2)
  agg2 = _sc_agg(g2.reshape(NC * N, HALF), src, dst, w, nsrc)
  return _tc_out(agg2, norm3, b2)


# speedup vs baseline: 1.0025x; 1.0018x over previous
import dataclasses
import functools

import jax
import jax.numpy as jnp
from jax import lax
from jax.experimental import pallas as pl
from jax.experimental.pallas import tpu as pltpu
from jax.experimental.pallas import tpu_sc as plsc

N = 10000
E = 160000
F_IN = 256
F_HID = 512
F_OUT = 256
NC = 2
NS = 16
LANES = 16
HALF = 128

EB = 128
NBLK = E // EB
KMAX = 80

ROWS = 624
ROWS_LAST = N - ROWS * (NS - 1)
DEGW = 128


def _sc_compiler_params():
  cp = pltpu.CompilerParams()
  if "needs_layout_passes" in pltpu.CompilerParams.__dataclass_fields__:
    cp = dataclasses.replace(cp, needs_layout_passes=False)
  return cp


def _vmesh():
  return plsc.VectorSubcoreMesh(core_axis_name="c", subcore_axis_name="s")


def _zero_fill(ref, rows, width):
  @pl.loop(0, rows)
  def _(i):
    for j in range(width // LANES):
      ref[i, pl.ds(j * LANES, LANES)] = jnp.zeros((LANES,), jnp.float32)


_ZCHUNKS = ((0, 128), (128, 128), (256, 128), (384, 128), (512, 112))


def _zero_spmem(acc_sp, zbuf, s, zsem):
  base = ROWS * s
  for off, n in _ZCHUNKS:
    pltpu.async_copy(zbuf.at[pl.ds(0, n)], acc_sp.at[pl.ds(base + off, n)],
                     zsem)

  @pl.when(s == NS - 1)
  def _():
    pltpu.async_copy(zbuf.at[pl.ds(0, ROWS_LAST - ROWS)],
                     acc_sp.at[pl.ds(ROWS * NS, ROWS_LAST - ROWS)], zsem)

  for off, n in _ZCHUNKS:
    pltpu.make_async_copy(zbuf.at[pl.ds(0, n)],
                          acc_sp.at[pl.ds(base + off, n)], zsem).wait()

  @pl.when(s == NS - 1)
  def _():
    pltpu.make_async_copy(zbuf.at[pl.ds(0, ROWS_LAST - ROWS)],
                          acc_sp.at[pl.ds(ROWS * NS, ROWS_LAST - ROWS)],
                          zsem).wait()


def _copy_out(acc_sp, out_hbm, c, s):
  row0 = ROWS * s
  pltpu.sync_copy(acc_sp.at[pl.ds(row0, ROWS)],
                  out_hbm.at[c].at[pl.ds(row0, ROWS)])

  @pl.when(s == NS - 1)
  def _():
    row1 = ROWS * NS
    pltpu.sync_copy(acc_sp.at[pl.ds(row1, ROWS_LAST - ROWS)],
                    out_hbm.at[c].at[pl.ds(row1, ROWS_LAST - ROWS)])


HROWS = 80


def _deg_body(src_hbm, dst_hbm, cnt_hbm, acc_sp,
              idx0, idx1, hist_v, ident_v, isem0, isem1):
  c = lax.axis_index("c")
  s = lax.axis_index("s")
  idx_vs = (idx0, idx1)
  isems = (isem0, isem1)

  _zero_fill(hist_v, HROWS, DEGW)
  for g in range(HROWS // LANES):
    ident_v[pl.ds(g * LANES, LANES)] = (
        lax.iota(jnp.int32, LANES) + g * LANES)

  @pl.when(s == 0)
  def _():
    pltpu.sync_copy(hist_v, acc_sp)
  plsc.subcore_barrier()

  def valid(k):
    return (k * NS + s) < NBLK

  def issue_idx(k, slot):
    sl = pl.ds((k * NS + s) * EB, EB)

    @pl.when(c == 0)
    def _():
      pltpu.async_copy(src_hbm.at[sl], idx_vs[slot], isems[slot])

    @pl.when(c == 1)
    def _():
      pltpu.async_copy(dst_hbm.at[sl], idx_vs[slot], isems[slot])

  def wait_idx(slot):
    pltpu.make_async_copy(src_hbm.at[pl.ds(0, EB)], idx_vs[slot],
                          isems[slot]).wait()

  issue_idx(0, 0)
  ones16 = jnp.ones((LANES,), jnp.float32)

  @pl.loop(0, KMAX, step=2)
  def _(k0):
    for dk in (0, 1):
      k = k0 + dk
      r, o = dk, 1 - dk

      @pl.when(valid(k + 1))
      def _():
        issue_idx(k + 1, o)

      @pl.when(valid(k))
      def _():
        wait_idx(r)
        for g in range(EB // LANES):
          idx16 = idx_vs[r][pl.ds(g * LANES, LANES)]
          row16 = lax.shift_right_logical(idx16, 7)
          col16 = lax.bitwise_and(idx16, 127)
          plsc.addupdate_scatter(hist_v, [row16, col16], ones16)

  pltpu.sync_copy(hist_v, acc_sp.at[ident_v], add=True)
  plsc.subcore_barrier()

  @pl.when(s == 0)
  def _():
    pltpu.sync_copy(acc_sp, cnt_hbm.at[c])


def _sc_degrees(src, dst):
  kern = pl.kernel(
      _deg_body,
      out_type=jax.ShapeDtypeStruct((NC, HROWS, DEGW), jnp.float32),
      mesh=_vmesh(),
      scratch_types=[
          pltpu.VMEM_SHARED((HROWS, DEGW), jnp.float32),
          pltpu.VMEM((EB,), jnp.int32),
          pltpu.VMEM((EB,), jnp.int32),
          pltpu.VMEM((HROWS, DEGW), jnp.float32),
          pltpu.VMEM((HROWS,), jnp.int32),
          pltpu.SemaphoreType.DMA,
          pltpu.SemaphoreType.DMA,
      ],
      compiler_params=_sc_compiler_params(),
  )
  return kern(src, dst)


def _agg_body(tbl_hbm, src_hbm, dst_hbm, w_hbm, scale_hbm, out_hbm, acc_sp,
              idx0, idx1, dst0, dst1, w0, w1, rows0, rows1, dstS, dstS2,
              scale_v, isem0, isem1, gsem0, gsem1, ssem):
  c = lax.axis_index("c")
  s = lax.axis_index("s")
  idx_vs = (idx0, idx1)
  dst_vs = (dst0, dst1)
  w_vs = (w0, w1)
  rows_vs = (rows0, rows1)
  isems = (isem0, isem1)
  gsems = (gsem0, gsem1)

  _zero_fill(rows0, EB, HALF)
  _zero_spmem(acc_sp, rows0, s, ssem)
  pltpu.sync_copy(scale_hbm, scale_v)
  plsc.subcore_barrier()

  def valid(k):
    return (k * NS + s) < NBLK

  def issue_idx(k, slot):
    sl = pl.ds((k * NS + s) * EB, EB)
    pltpu.async_copy(src_hbm.at[sl], idx_vs[slot], isems[slot])
    pltpu.async_copy(dst_hbm.at[sl], dst_vs[slot], isems[slot])
    pltpu.async_copy(w_hbm.at[sl], w_vs[slot], isems[slot])

  def wait_idx(slot):
    pltpu.make_async_copy(src_hbm.at[pl.ds(0, EB)], idx_vs[slot],
                          isems[slot]).wait()
    pltpu.make_async_copy(dst_hbm.at[pl.ds(0, EB)], dst_vs[slot],
                          isems[slot]).wait()
    pltpu.make_async_copy(w_hbm.at[pl.ds(0, EB)], w_vs[slot],
                          isems[slot]).wait()

  def transform_idx(slot):
    for g in range(EB // LANES):
      gsl = pl.ds(g * LANES, LANES)
      s16 = idx_vs[slot][gsl]
      idx_vs[slot][gsl] = s16 + s16 + c

  def fold_weight(slot):
    for g in range(EB // LANES):
      gsl = pl.ds(g * LANES, LANES)
      s16 = lax.shift_right_logical(idx_vs[slot][gsl], 1)
      w_vs[slot][gsl] = w_vs[slot][gsl] * plsc.load_gather(scale_v, [s16])

  HB = EB // 2

  def issue_gather(slot):
    pltpu.async_copy(tbl_hbm.at[idx_vs[slot].at[pl.ds(0, HB)]],
                     rows_vs[slot].at[pl.ds(0, HB)], gsems[slot])
    pltpu.async_copy(tbl_hbm.at[idx_vs[slot].at[pl.ds(HB, HB)]],
                     rows_vs[slot].at[pl.ds(HB, HB)], gsems[slot])

  def wait_gather(slot):
    pltpu.make_async_copy(tbl_hbm.at[idx_vs[slot].at[pl.ds(0, HB)]],
                          rows_vs[slot].at[pl.ds(0, HB)], gsems[slot]).wait()
    pltpu.make_async_copy(tbl_hbm.at[idx_vs[slot].at[pl.ds(HB, HB)]],
                          rows_vs[slot].at[pl.ds(HB, HB)], gsems[slot]).wait()

  issue_idx(0, 0)
  wait_idx(0)
  transform_idx(0)
  issue_gather(0)
  fold_weight(0)
  issue_idx(1, 1)

  @pl.loop(0, KMAX, step=2)
  def _(k0):
    for dk in (0, 1):
      k = k0 + dk
      r, o = dk, 1 - dk

      @pl.when(jnp.logical_and(k >= 1, valid(k - 1)))
      def _():
        pltpu.make_async_copy(rows_vs[o].at[pl.ds(0, HB)],
                              acc_sp.at[dstS], ssem).wait()
        pltpu.make_async_copy(rows_vs[o].at[pl.ds(HB, HB)],
                              acc_sp.at[dstS2], ssem).wait()

      @pl.when(valid(k + 1))
      def _():
        wait_idx(o)
        transform_idx(o)
        issue_gather(o)
        fold_weight(o)

      @pl.when(valid(k))
      def _():
        wait_gather(r)

        for g in range(EB // LANES):
          gsl = pl.ds(g * LANES, LANES)
          if g < HB // LANES:
            dstS[gsl] = dst_vs[r][gsl]
          else:
            dstS2[pl.ds(g * LANES - HB, LANES)] = dst_vs[r][gsl]

        @plsc.parallel_loop(0, EB, unroll=4)
        def _(i):
          wspl = plsc.load_gather(w_vs[r], [jnp.broadcast_to(i, (LANES,))])
          for j in range(HALF // LANES):
            jsl = pl.ds(j * LANES, LANES)
            rows_vs[r][i, jsl] = rows_vs[r][i, jsl] * wspl

        pltpu.async_copy(rows_vs[r].at[pl.ds(0, HB)], acc_sp.at[dstS],
                         ssem, add=True)
        pltpu.async_copy(rows_vs[r].at[pl.ds(HB, HB)], acc_sp.at[dstS2],
                         ssem, add=True)

      @pl.when(valid(k + 2))
      def _():
        issue_idx(k + 2, r)

  plsc.subcore_barrier()
  _copy_out(acc_sp, out_hbm, c, s)


def _sc_agg(table, src, dst, w, scale):
  kern = pl.kernel(
      _agg_body,
      out_type=jax.ShapeDtypeStruct((NC, N, HALF), jnp.float32),
      mesh=_vmesh(),
      scratch_types=[
          pltpu.VMEM_SHARED((N, HALF), jnp.float32),
          pltpu.VMEM((EB,), jnp.int32),
          pltpu.VMEM((EB,), jnp.int32),
          pltpu.VMEM((EB,), jnp.int32),
          pltpu.VMEM((EB,), jnp.int32),
          pltpu.VMEM((EB,), jnp.float32),
          pltpu.VMEM((EB,), jnp.float32),
          pltpu.VMEM((EB, HALF), jnp.float32),
          pltpu.VMEM((EB, HALF), jnp.float32),
          pltpu.VMEM((EB // 2,), jnp.int32),
          pltpu.VMEM((EB // 2,), jnp.int32),
          pltpu.VMEM((N,), jnp.float32),
          pltpu.SemaphoreType.DMA,
          pltpu.SemaphoreType.DMA,
          pltpu.SemaphoreType.DMA,
          pltpu.SemaphoreType.DMA,
          pltpu.SemaphoreType.DMA,
      ],
      compiler_params=_sc_compiler_params(),
  )
  return kern(table, src, dst, w, scale)


def _norm_body(cnt_ref, out_ref):
  flat = cnt_ref[...].reshape(NC, HROWS * DEGW)[:, :N]
  out_ref[...] = lax.rsqrt(jnp.maximum(flat, 1.0))[:, :, None]


def _tc_norm(cnt):
  return pl.pallas_call(
      _norm_body,
      out_shape=jax.ShapeDtypeStruct((NC, N, 1), jnp.float32),
  )(cnt)


_MM_BLK = 1000


def _mm_body(agg_ref, ndst_ref, w1_ref, b1_ref, w2_ref, out_ref):
  a = jnp.concatenate([agg_ref[0], agg_ref[1]], axis=-1)
  a = a * ndst_ref[0]
  h = jnp.dot(a, w1_ref[...], preferred_element_type=jnp.float32,
              precision=lax.Precision.DEFAULT)
  h = jnp.maximum(h + b1_ref[...][None, :], 0.0)
  out_ref[...] = jnp.dot(h, w2_ref[...], preferred_element_type=jnp.float32,
                         precision=lax.Precision.DEFAULT)


def _tc_mm(agg, norm3, W1, b1, W2):
  return pl.pallas_call(
      _mm_body,
      grid=(N // _MM_BLK,),
      in_specs=[
          pl.BlockSpec((NC, _MM_BLK, HALF), lambda i: (0, i, 0)),
          pl.BlockSpec((1, _MM_BLK, 1), lambda i: (1, i, 0)),
          pl.BlockSpec((F_IN, F_HID), lambda i: (0, 0)),
          pl.BlockSpec((F_HID,), lambda i: (0,)),
          pl.BlockSpec((F_HID, F_OUT), lambda i: (0, 0)),
      ],
      out_specs=pl.BlockSpec((_MM_BLK, F_OUT), lambda i: (i, 0)),
      out_shape=jax.ShapeDtypeStruct((N, F_OUT), jnp.float32),
  )(agg, norm3, W1, b1, W2)


def _out_body(agg_ref, ndst_ref, b2_ref, out_ref):
  o = jnp.concatenate([agg_ref[0], agg_ref[1]], axis=-1)
  out_ref[...] = o * ndst_ref[0] + b2_ref[...][None, :]


def _tc_out(agg, norm3, b2):
  return pl.pallas_call(
      _out_body,
      grid=(N // _MM_BLK,),
      in_specs=[
          pl.BlockSpec((NC, _MM_BLK, HALF), lambda i: (0, i, 0)),
          pl.BlockSpec((1, _MM_BLK, 1), lambda i: (1, i, 0)),
          pl.BlockSpec((F_OUT,), lambda i: (0,)),
      ],
      out_specs=pl.BlockSpec((_MM_BLK, F_OUT), lambda i: (i, 0)),
      out_shape=jax.ShapeDtypeStruct((N, F_OUT), jnp.float32),
  )(agg, norm3, b2)


def kernel(node_feats, edge_index, edge_weight, W1, b1, W2, b2):
  ei = edge_index.astype(jnp.int32)
  src = ei[0]
  dst = ei[1]
  w = edge_weight.astype(jnp.float32)

  cnt = _sc_degrees(src, dst)
  norm3 = _tc_norm(cnt)
  nsrc = norm3[0, :, 0]

  agg1 = _sc_agg(node_feats.reshape(NC * N, HALF), src, dst, w, nsrc)
  g2 = _tc_mm(agg1, norm3, W1, b1, W2)
  agg2 = _sc_agg(g2.reshape(NC * N, HALF), src, dst, w, nsrc)
  return _tc_out(agg2, norm3, b2)


# speedup vs baseline: 1.0095x; 1.0070x over previous
import dataclasses
import functools

import jax
import jax.numpy as jnp
from jax import lax
from jax.experimental import pallas as pl
from jax.experimental.pallas import tpu as pltpu
from jax.experimental.pallas import tpu_sc as plsc

N = 10000
E = 160000
F_IN = 256
F_HID = 512
F_OUT = 256
NC = 2
NS = 16
LANES = 16
HALF = 128

EB = 128
NBLK = E // EB
KMAX = 80

ROWS = 624
ROWS_LAST = N - ROWS * (NS - 1)
DEGW = 128


def _sc_compiler_params():
  cp = pltpu.CompilerParams()
  if "needs_layout_passes" in pltpu.CompilerParams.__dataclass_fields__:
    cp = dataclasses.replace(cp, needs_layout_passes=False)
  return cp


def _vmesh():
  return plsc.VectorSubcoreMesh(core_axis_name="c", subcore_axis_name="s")


def _zero_fill(ref, rows, width):
  @pl.loop(0, rows)
  def _(i):
    for j in range(width // LANES):
      ref[i, pl.ds(j * LANES, LANES)] = jnp.zeros((LANES,), jnp.float32)


_ZCHUNKS = ((0, 128), (128, 128), (256, 128), (384, 128), (512, 112))


def _zero_spmem(acc_sp, zbuf, s, zsem):
  base = ROWS * s
  for off, n in _ZCHUNKS:
    pltpu.async_copy(zbuf.at[pl.ds(0, n)], acc_sp.at[pl.ds(base + off, n)],
                     zsem)

  @pl.when(s == NS - 1)
  def _():
    pltpu.async_copy(zbuf.at[pl.ds(0, ROWS_LAST - ROWS)],
                     acc_sp.at[pl.ds(ROWS * NS, ROWS_LAST - ROWS)], zsem)

  for off, n in _ZCHUNKS:
    pltpu.make_async_copy(zbuf.at[pl.ds(0, n)],
                          acc_sp.at[pl.ds(base + off, n)], zsem).wait()

  @pl.when(s == NS - 1)
  def _():
    pltpu.make_async_copy(zbuf.at[pl.ds(0, ROWS_LAST - ROWS)],
                          acc_sp.at[pl.ds(ROWS * NS, ROWS_LAST - ROWS)],
                          zsem).wait()


def _copy_out(acc_sp, out_hbm, c, s):
  row0 = ROWS * s
  pltpu.sync_copy(acc_sp.at[pl.ds(row0, ROWS)],
                  out_hbm.at[c].at[pl.ds(row0, ROWS)])

  @pl.when(s == NS - 1)
  def _():
    row1 = ROWS * NS
    pltpu.sync_copy(acc_sp.at[pl.ds(row1, ROWS_LAST - ROWS)],
                    out_hbm.at[c].at[pl.ds(row1, ROWS_LAST - ROWS)])


HROWS = 80


def _deg_body(src_hbm, dst_hbm, cnt_hbm, acc_sp,
              idx0, idx1, hist_v, ident_v, isem0, isem1):
  c = lax.axis_index("c")
  s = lax.axis_index("s")
  idx_vs = (idx0, idx1)
  isems = (isem0, isem1)

  _zero_fill(hist_v, HROWS, DEGW)
  for g in range(HROWS // LANES):
    ident_v[pl.ds(g * LANES, LANES)] = (
        lax.iota(jnp.int32, LANES) + g * LANES)

  @pl.when(s == 0)
  def _():
    pltpu.sync_copy(hist_v, acc_sp)
  plsc.subcore_barrier()

  def valid(k):
    return (k * NS + s) < NBLK

  def issue_idx(k, slot):
    sl = pl.ds((k * NS + s) * EB, EB)

    @pl.when(c == 0)
    def _():
      pltpu.async_copy(src_hbm.at[sl], idx_vs[slot], isems[slot])

    @pl.when(c == 1)
    def _():
      pltpu.async_copy(dst_hbm.at[sl], idx_vs[slot], isems[slot])

  def wait_idx(slot):
    pltpu.make_async_copy(src_hbm.at[pl.ds(0, EB)], idx_vs[slot],
                          isems[slot]).wait()

  issue_idx(0, 0)
  ones16 = jnp.ones((LANES,), jnp.float32)

  @pl.loop(0, KMAX, step=2)
  def _(k0):
    for dk in (0, 1):
      k = k0 + dk
      r, o = dk, 1 - dk

      @pl.when(valid(k + 1))
      def _():
        issue_idx(k + 1, o)

      @pl.when(valid(k))
      def _():
        wait_idx(r)
        for g in range(EB // LANES):
          idx16 = idx_vs[r][pl.ds(g * LANES, LANES)]
          row16 = lax.shift_right_logical(idx16, 7)
          col16 = lax.bitwise_and(idx16, 127)
          plsc.addupdate_scatter(hist_v, [row16, col16], ones16)

  pltpu.sync_copy(hist_v, acc_sp.at[ident_v], add=True)
  plsc.subcore_barrier()

  @pl.when(s == 0)
  def _():
    pltpu.sync_copy(acc_sp, cnt_hbm.at[c])


def _sc_degrees(src, dst):
  kern = pl.kernel(
      _deg_body,
      out_type=jax.ShapeDtypeStruct((NC, HROWS, DEGW), jnp.float32),
      mesh=_vmesh(),
      scratch_types=[
          pltpu.VMEM_SHARED((HROWS, DEGW), jnp.float32),
          pltpu.VMEM((EB,), jnp.int32),
          pltpu.VMEM((EB,), jnp.int32),
          pltpu.VMEM((HROWS, DEGW), jnp.float32),
          pltpu.VMEM((HROWS,), jnp.int32),
          pltpu.SemaphoreType.DMA,
          pltpu.SemaphoreType.DMA,
      ],
      compiler_params=_sc_compiler_params(),
  )
  return kern(src, dst)


def _agg_body(tbl_hbm, src_hbm, dst_hbm, w_hbm, scale_hbm, out_hbm, acc_sp,
              idx0, idx1, dst0, dst1, w0, w1, rows0, rows1, dstS, dstS2,
              scale_v, scale_sp, isem0, isem1, gsem0, gsem1, ssem):
  c = lax.axis_index("c")
  s = lax.axis_index("s")
  idx_vs = (idx0, idx1)
  dst_vs = (dst0, dst1)
  w_vs = (w0, w1)
  rows_vs = (rows0, rows1)
  isems = (isem0, isem1)
  gsems = (gsem0, gsem1)

  _zero_fill(rows0, EB, HALF)
  _zero_spmem(acc_sp, rows0, s, ssem)

  @pl.when(s == 0)
  def _():
    pltpu.sync_copy(scale_hbm, scale_sp)
  plsc.subcore_barrier()
  pltpu.sync_copy(scale_sp, scale_v)

  def valid(k):
    return (k * NS + s) < NBLK

  def issue_idx(k, slot):
    sl = pl.ds((k * NS + s) * EB, EB)
    pltpu.async_copy(src_hbm.at[sl], idx_vs[slot], isems[slot])
    pltpu.async_copy(dst_hbm.at[sl], dst_vs[slot], isems[slot])
    pltpu.async_copy(w_hbm.at[sl], w_vs[slot], isems[slot])

  def wait_idx(slot):
    pltpu.make_async_copy(src_hbm.at[pl.ds(0, EB)], idx_vs[slot],
                          isems[slot]).wait()
    pltpu.make_async_copy(dst_hbm.at[pl.ds(0, EB)], dst_vs[slot],
                          isems[slot]).wait()
    pltpu.make_async_copy(w_hbm.at[pl.ds(0, EB)], w_vs[slot],
                          isems[slot]).wait()

  def transform_idx(slot):
    for g in range(EB // LANES):
      gsl = pl.ds(g * LANES, LANES)
      s16 = idx_vs[slot][gsl]
      idx_vs[slot][gsl] = s16 + s16 + c

  def fold_weight(slot):
    for g in range(EB // LANES):
      gsl = pl.ds(g * LANES, LANES)
      s16 = lax.shift_right_logical(idx_vs[slot][gsl], 1)
      w_vs[slot][gsl] = w_vs[slot][gsl] * plsc.load_gather(scale_v, [s16])

  HB = EB // 2

  def issue_gather(slot):
    pltpu.async_copy(tbl_hbm.at[idx_vs[slot].at[pl.ds(0, HB)]],
                     rows_vs[slot].at[pl.ds(0, HB)], gsems[slot])
    pltpu.async_copy(tbl_hbm.at[idx_vs[slot].at[pl.ds(HB, HB)]],
                     rows_vs[slot].at[pl.ds(HB, HB)], gsems[slot])

  def wait_gather(slot):
    pltpu.make_async_copy(tbl_hbm.at[idx_vs[slot].at[pl.ds(0, HB)]],
                          rows_vs[slot].at[pl.ds(0, HB)], gsems[slot]).wait()
    pltpu.make_async_copy(tbl_hbm.at[idx_vs[slot].at[pl.ds(HB, HB)]],
                          rows_vs[slot].at[pl.ds(HB, HB)], gsems[slot]).wait()

  issue_idx(0, 0)
  wait_idx(0)
  transform_idx(0)
  issue_gather(0)
  fold_weight(0)
  issue_idx(1, 1)

  @pl.loop(0, KMAX, step=2)
  def _(k0):
    for dk in (0, 1):
      k = k0 + dk
      r, o = dk, 1 - dk

      @pl.when(jnp.logical_and(k >= 1, valid(k - 1)))
      def _():
        pltpu.make_async_copy(rows_vs[o].at[pl.ds(0, HB)],
                              acc_sp.at[dstS], ssem).wait()
        pltpu.make_async_copy(rows_vs[o].at[pl.ds(HB, HB)],
                              acc_sp.at[dstS2], ssem).wait()

      @pl.when(valid(k + 1))
      def _():
        wait_idx(o)
        transform_idx(o)
        issue_gather(o)
        fold_weight(o)

      @pl.when(valid(k))
      def _():
        wait_gather(r)

        for g in range(EB // LANES):
          gsl = pl.ds(g * LANES, LANES)
          if g < HB // LANES:
            dstS[gsl] = dst_vs[r][gsl]
          else:
            dstS2[pl.ds(g * LANES - HB, LANES)] = dst_vs[r][gsl]

        @plsc.parallel_loop(0, EB, unroll=4)
        def _(i):
          wspl = plsc.load_gather(w_vs[r], [jnp.broadcast_to(i, (LANES,))])
          for j in range(HALF // LANES):
            jsl = pl.ds(j * LANES, LANES)
            rows_vs[r][i, jsl] = rows_vs[r][i, jsl] * wspl

        pltpu.async_copy(rows_vs[r].at[pl.ds(0, HB)], acc_sp.at[dstS],
                         ssem, add=True)
        pltpu.async_copy(rows_vs[r].at[pl.ds(HB, HB)], acc_sp.at[dstS2],
                         ssem, add=True)

      @pl.when(valid(k + 2))
      def _():
        issue_idx(k + 2, r)

  plsc.subcore_barrier()
  _copy_out(acc_sp, out_hbm, c, s)


def _sc_agg(table, src, dst, w, scale):
  kern = pl.kernel(
      _agg_body,
      out_type=jax.ShapeDtypeStruct((NC, N, HALF), jnp.float32),
      mesh=_vmesh(),
      scratch_types=[
          pltpu.VMEM_SHARED((N, HALF), jnp.float32),
          pltpu.VMEM((EB,), jnp.int32),
          pltpu.VMEM((EB,), jnp.int32),
          pltpu.VMEM((EB,), jnp.int32),
          pltpu.VMEM((EB,), jnp.int32),
          pltpu.VMEM((EB,), jnp.float32),
          pltpu.VMEM((EB,), jnp.float32),
          pltpu.VMEM((EB, HALF), jnp.float32),
          pltpu.VMEM((EB, HALF), jnp.float32),
          pltpu.VMEM((EB // 2,), jnp.int32),
          pltpu.VMEM((EB // 2,), jnp.int32),
          pltpu.VMEM((N,), jnp.float32),
          pltpu.VMEM_SHARED((N,), jnp.float32),
          pltpu.SemaphoreType.DMA,
          pltpu.SemaphoreType.DMA,
          pltpu.SemaphoreType.DMA,
          pltpu.SemaphoreType.DMA,
          pltpu.SemaphoreType.DMA,
      ],
      compiler_params=_sc_compiler_params(),
  )
  return kern(table, src, dst, w, scale)


def _norm_body(cnt_ref, out_ref):
  flat = cnt_ref[...].reshape(NC, HROWS * DEGW)[:, :N]
  out_ref[...] = lax.rsqrt(jnp.maximum(flat, 1.0))[:, :, None]


def _tc_norm(cnt):
  return pl.pallas_call(
      _norm_body,
      out_shape=jax.ShapeDtypeStruct((NC, N, 1), jnp.float32),
  )(cnt)


_MM_BLK = 1000


def _mm_body(agg_ref, ndst_ref, w1_ref, b1_ref, w2_ref, out_ref):
  a = jnp.concatenate([agg_ref[0], agg_ref[1]], axis=-1)
  a = a * ndst_ref[0]
  h = jnp.dot(a, w1_ref[...], preferred_element_type=jnp.float32,
              precision=lax.Precision.DEFAULT)
  h = jnp.maximum(h + b1_ref[...][None, :], 0.0)
  out_ref[...] = jnp.dot(h, w2_ref[...], preferred_element_type=jnp.float32,
                         precision=lax.Precision.DEFAULT)


def _tc_mm(agg, norm3, W1, b1, W2):
  return pl.pallas_call(
      _mm_body,
      grid=(N // _MM_BLK,),
      in_specs=[
          pl.BlockSpec((NC, _MM_BLK, HALF), lambda i: (0, i, 0)),
          pl.BlockSpec((1, _MM_BLK, 1), lambda i: (1, i, 0)),
          pl.BlockSpec((F_IN, F_HID), lambda i: (0, 0)),
          pl.BlockSpec((F_HID,), lambda i: (0,)),
          pl.BlockSpec((F_HID, F_OUT), lambda i: (0, 0)),
      ],
      out_specs=pl.BlockSpec((_MM_BLK, F_OUT), lambda i: (i, 0)),
      out_shape=jax.ShapeDtypeStruct((N, F_OUT), jnp.float32),
  )(agg, norm3, W1, b1, W2)


def _out_body(agg_ref, ndst_ref, b2_ref, out_ref):
  o = jnp.concatenate([agg_ref[0], agg_ref[1]], axis=-1)
  out_ref[...] = o * ndst_ref[0] + b2_ref[...][None, :]


def _tc_out(agg, norm3, b2):
  return pl.pallas_call(
      _out_body,
      grid=(N // _MM_BLK,),
      in_specs=[
          pl.BlockSpec((NC, _MM_BLK, HALF), lambda i: (0, i, 0)),
          pl.BlockSpec((1, _MM_BLK, 1), lambda i: (1, i, 0)),
          pl.BlockSpec((F_OUT,), lambda i: (0,)),
      ],
      out_specs=pl.BlockSpec((_MM_BLK, F_OUT), lambda i: (i, 0)),
      out_shape=jax.ShapeDtypeStruct((N, F_OUT), jnp.float32),
  )(agg, norm3, b2)


def kernel(node_feats, edge_index, edge_weight, W1, b1, W2, b2):
  ei = edge_index.astype(jnp.int32)
  src = ei[0]
  dst = ei[1]
  w = edge_weight.astype(jnp.float32)

  cnt = _sc_degrees(src, dst)
  norm3 = _tc_norm(cnt)
  nsrc = norm3[0, :, 0]

  agg1 = _sc_agg(node_feats.reshape(NC * N, HALF), src, dst, w, nsrc)
  g2 = _tc_mm(agg1, norm3, W1, b1, W2)
  agg2 = _sc_agg(g2.reshape(NC * N, HALF), src, dst, w, nsrc)
  return _tc_out(agg2, norm3, b2)


# speedup vs baseline: 1.0218x; 1.0122x over previous
import dataclasses

import jax
import jax.numpy as jnp
from jax import lax
from jax.experimental import pallas as pl
from jax.experimental.pallas import tpu as pltpu
from jax.experimental.pallas import tpu_sc as plsc

N = 10000
E = 160000
F_IN = 256
F_HID = 512
F_OUT = 256
NC = 2
NS = 16
LANES = 16
HALF = 128

EB = 128
NBLK = E // EB
KMAX = 80

ROWS = 624
ROWS_LAST = N - ROWS * (NS - 1)
DEGW = 128


def _sc_compiler_params():
  cp = pltpu.CompilerParams()
  if "needs_layout_passes" in pltpu.CompilerParams.__dataclass_fields__:
    cp = dataclasses.replace(cp, needs_layout_passes=False)
  return cp


def _vmesh():
  return plsc.VectorSubcoreMesh(core_axis_name="c", subcore_axis_name="s")


def _zero_fill(ref, rows, width):
  @pl.loop(0, rows)
  def _(i):
    for j in range(width // LANES):
      ref[i, pl.ds(j * LANES, LANES)] = jnp.zeros((LANES,), jnp.float32)


_ZCHUNKS = ((0, 128), (128, 128), (256, 128), (384, 128), (512, 112))


def _zero_spmem(acc_sp, zbuf, s, zsem):
  base = ROWS * s
  for off, n in _ZCHUNKS:
    pltpu.async_copy(zbuf.at[pl.ds(0, n)], acc_sp.at[pl.ds(base + off, n)],
                     zsem)

  @pl.when(s == NS - 1)
  def _():
    pltpu.async_copy(zbuf.at[pl.ds(0, ROWS_LAST - ROWS)],
                     acc_sp.at[pl.ds(ROWS * NS, ROWS_LAST - ROWS)], zsem)

  for off, n in _ZCHUNKS:
    pltpu.make_async_copy(zbuf.at[pl.ds(0, n)],
                          acc_sp.at[pl.ds(base + off, n)], zsem).wait()

  @pl.when(s == NS - 1)
  def _():
    pltpu.make_async_copy(zbuf.at[pl.ds(0, ROWS_LAST - ROWS)],
                          acc_sp.at[pl.ds(ROWS * NS, ROWS_LAST - ROWS)],
                          zsem).wait()


def _copy_out(acc_sp, out_hbm, c, s):
  row0 = ROWS * s
  pltpu.sync_copy(acc_sp.at[pl.ds(row0, ROWS)],
                  out_hbm.at[c].at[pl.ds(row0, ROWS)])

  @pl.when(s == NS - 1)
  def _():
    row1 = ROWS * NS
    pltpu.sync_copy(acc_sp.at[pl.ds(row1, ROWS_LAST - ROWS)],
                    out_hbm.at[c].at[pl.ds(row1, ROWS_LAST - ROWS)])


HROWS = 80


def _deg_body(src_hbm, dst_hbm, cnt_hbm, acc_sp,
              idx0, idx1, hist_v, ident_v, isem0, isem1):
  c = lax.axis_index("c")
  s = lax.axis_index("s")
  idx_vs = (idx0, idx1)
  isems = (isem0, isem1)

  _zero_fill(hist_v, HROWS, DEGW)
  for g in range(HROWS // LANES):
    ident_v[pl.ds(g * LANES, LANES)] = (
        lax.iota(jnp.int32, LANES) + g * LANES)

  @pl.when(s == 0)
  def _():
    pltpu.sync_copy(hist_v, acc_sp)
  plsc.subcore_barrier()

  def valid(k):
    return (k * NS + s) < NBLK

  def issue_idx(k, slot):
    sl = pl.ds((k * NS + s) * EB, EB)

    @pl.when(c == 0)
    def _():
      pltpu.async_copy(src_hbm.at[sl], idx_vs[slot], isems[slot])

    @pl.when(c == 1)
    def _():
      pltpu.async_copy(dst_hbm.at[sl], idx_vs[slot], isems[slot])

  def wait_idx(slot):
    pltpu.make_async_copy(src_hbm.at[pl.ds(0, EB)], idx_vs[slot],
                          isems[slot]).wait()

  issue_idx(0, 0)
  ones16 = jnp.ones((LANES,), jnp.float32)

  @pl.loop(0, KMAX, step=2)
  def _(k0):
    for dk in (0, 1):
      k = k0 + dk
      r, o = dk, 1 - dk

      @pl.when(valid(k + 1))
      def _():
        issue_idx(k + 1, o)

      @pl.when(valid(k))
      def _():
        wait_idx(r)
        for g in range(EB // LANES):
          idx16 = idx_vs[r][pl.ds(g * LANES, LANES)]
          row16 = lax.shift_right_logical(idx16, 7)
          col16 = lax.bitwise_and(idx16, 127)
          plsc.addupdate_scatter(hist_v, [row16, col16], ones16)

  pltpu.sync_copy(hist_v, acc_sp.at[ident_v], add=True)
  plsc.subcore_barrier()

  @pl.when(s == 0)
  def _():
    pltpu.sync_copy(acc_sp, cnt_hbm.at[c])


def _sc_degrees(src, dst):
  kern = pl.kernel(
      _deg_body,
      out_type=jax.ShapeDtypeStruct((NC, HROWS, DEGW), jnp.float32),
      mesh=_vmesh(),
      scratch_types=[
          pltpu.VMEM_SHARED((HROWS, DEGW), jnp.float32),
          pltpu.VMEM((EB,), jnp.int32),
          pltpu.VMEM((EB,), jnp.int32),
          pltpu.VMEM((HROWS, DEGW), jnp.float32),
          pltpu.VMEM((HROWS,), jnp.int32),
          pltpu.SemaphoreType.DMA,
          pltpu.SemaphoreType.DMA,
      ],
      compiler_params=_sc_compiler_params(),
  )
  return kern(src, dst)


def _agg_body(tbl_hbm, src_hbm, dst_hbm, w_hbm, scale_hbm, out_hbm, acc_sp,
              idx0, idx1, dst0, dst1, w0, w1, rows0, rows1, dstS, dstS2,
              scale_v, scale_sp, isem0, isem1, gsem0, gsem1, ssem):
  c = lax.axis_index("c")
  s = lax.axis_index("s")
  idx_vs = (idx0, idx1)
  dst_vs = (dst0, dst1)
  w_vs = (w0, w1)
  rows_vs = (rows0, rows1)
  isems = (isem0, isem1)
  gsems = (gsem0, gsem1)

  _zero_fill(rows0, EB, HALF)
  _zero_spmem(acc_sp, rows0, s, ssem)

  @pl.when(s == 0)
  def _():
    pltpu.sync_copy(scale_hbm, scale_sp)
  plsc.subcore_barrier()
  pltpu.sync_copy(scale_sp, scale_v)

  def valid(k):
    return (k * NS + s) < NBLK

  def issue_idx(k, slot):
    sl = pl.ds((k * NS + s) * EB, EB)
    pltpu.async_copy(src_hbm.at[sl], idx_vs[slot], isems[slot])
    pltpu.async_copy(dst_hbm.at[sl], dst_vs[slot], isems[slot])
    pltpu.async_copy(w_hbm.at[sl], w_vs[slot], isems[slot])

  def wait_idx(slot):
    pltpu.make_async_copy(src_hbm.at[pl.ds(0, EB)], idx_vs[slot],
                          isems[slot]).wait()
    pltpu.make_async_copy(dst_hbm.at[pl.ds(0, EB)], dst_vs[slot],
                          isems[slot]).wait()
    pltpu.make_async_copy(w_hbm.at[pl.ds(0, EB)], w_vs[slot],
                          isems[slot]).wait()

  def transform_idx(slot):
    for g in range(EB // LANES):
      gsl = pl.ds(g * LANES, LANES)
      s16 = idx_vs[slot][gsl]
      idx_vs[slot][gsl] = s16 + s16 + c

  def fold_weight(slot):
    for g in range(EB // LANES):
      gsl = pl.ds(g * LANES, LANES)
      s16 = lax.shift_right_logical(idx_vs[slot][gsl], 1)
      w_vs[slot][gsl] = w_vs[slot][gsl] * plsc.load_gather(scale_v, [s16])

  HB = EB // 2

  def issue_gather(slot):
    pltpu.async_copy(tbl_hbm.at[idx_vs[slot].at[pl.ds(0, HB)]],
                     rows_vs[slot].at[pl.ds(0, HB)], gsems[slot])
    pltpu.async_copy(tbl_hbm.at[idx_vs[slot].at[pl.ds(HB, HB)]],
                     rows_vs[slot].at[pl.ds(HB, HB)], gsems[slot])

  def wait_gather(slot):
    pltpu.make_async_copy(tbl_hbm.at[idx_vs[slot].at[pl.ds(0, HB)]],
                          rows_vs[slot].at[pl.ds(0, HB)], gsems[slot]).wait()
    pltpu.make_async_copy(tbl_hbm.at[idx_vs[slot].at[pl.ds(HB, HB)]],
                          rows_vs[slot].at[pl.ds(HB, HB)], gsems[slot]).wait()

  issue_idx(0, 0)
  wait_idx(0)
  transform_idx(0)
  issue_gather(0)
  fold_weight(0)
  issue_idx(1, 1)

  @pl.loop(0, KMAX, step=2)
  def _(k0):
    for dk in (0, 1):
      k = k0 + dk
      r, o = dk, 1 - dk

      @pl.when(jnp.logical_and(k >= 1, valid(k - 1)))
      def _():
        pltpu.make_async_copy(rows_vs[o].at[pl.ds(0, HB)],
                              acc_sp.at[dstS], ssem).wait()
        pltpu.make_async_copy(rows_vs[o].at[pl.ds(HB, HB)],
                              acc_sp.at[dstS2], ssem).wait()

      @pl.when(valid(k + 1))
      def _():
        wait_idx(o)
        transform_idx(o)
        issue_gather(o)
        fold_weight(o)

      @pl.when(valid(k))
      def _():
        wait_gather(r)

        for g in range(EB // LANES):
          gsl = pl.ds(g * LANES, LANES)
          if g < HB // LANES:
            dstS[gsl] = dst_vs[r][gsl]
          else:
            dstS2[pl.ds(g * LANES - HB, LANES)] = dst_vs[r][gsl]

        @plsc.parallel_loop(0, EB, unroll=4)
        def _(i):
          wspl = plsc.load_gather(w_vs[r], [jnp.broadcast_to(i, (LANES,))])
          for j in range(HALF // LANES):
            jsl = pl.ds(j * LANES, LANES)
            rows_vs[r][i, jsl] = rows_vs[r][i, jsl] * wspl

        pltpu.async_copy(rows_vs[r].at[pl.ds(0, HB)], acc_sp.at[dstS],
                         ssem, add=True)
        pltpu.async_copy(rows_vs[r].at[pl.ds(HB, HB)], acc_sp.at[dstS2],
                         ssem, add=True)

      @pl.when(valid(k + 2))
      def _():
        issue_idx(k + 2, r)

  plsc.subcore_barrier()
  _copy_out(acc_sp, out_hbm, c, s)


def _sc_agg(table, src, dst, w, scale):
  kern = pl.kernel(
      _agg_body,
      out_type=jax.ShapeDtypeStruct((NC, N, HALF), jnp.float32),
      mesh=_vmesh(),
      scratch_types=[
          pltpu.VMEM_SHARED((N, HALF), jnp.float32),
          pltpu.VMEM((EB,), jnp.int32),
          pltpu.VMEM((EB,), jnp.int32),
          pltpu.VMEM((EB,), jnp.int32),
          pltpu.VMEM((EB,), jnp.int32),
          pltpu.VMEM((EB,), jnp.float32),
          pltpu.VMEM((EB,), jnp.float32),
          pltpu.VMEM((EB, HALF), jnp.float32),
          pltpu.VMEM((EB, HALF), jnp.float32),
          pltpu.VMEM((EB // 2,), jnp.int32),
          pltpu.VMEM((EB // 2,), jnp.int32),
          pltpu.VMEM((N,), jnp.float32),
          pltpu.VMEM_SHARED((N,), jnp.float32),
          pltpu.SemaphoreType.DMA,
          pltpu.SemaphoreType.DMA,
          pltpu.SemaphoreType.DMA,
          pltpu.SemaphoreType.DMA,
          pltpu.SemaphoreType.DMA,
      ],
      compiler_params=_sc_compiler_params(),
  )
  return kern(table, src, dst, w, scale)


def _norm_body(cnt_ref, out_ref):
  flat = cnt_ref[...].reshape(NC, HROWS * DEGW)[:, :N]
  out_ref[...] = lax.rsqrt(jnp.maximum(flat, 1.0))[:, :, None]


def _tc_norm(cnt):
  return pl.pallas_call(
      _norm_body,
      out_shape=jax.ShapeDtypeStruct((NC, N, 1), jnp.float32),
  )(cnt)


_MM_BLK = 1000


def _mm_body(agg_ref, ndst_ref, w1_ref, b1_ref, w2_ref, out_ref):
  a = jnp.concatenate([agg_ref[0], agg_ref[1]], axis=-1)
  a = a * ndst_ref[0]
  h = jnp.dot(a, w1_ref[...], preferred_element_type=jnp.float32,
              precision=lax.Precision.DEFAULT)
  h = jnp.maximum(h + b1_ref[...][None, :], 0.0)
  out_ref[...] = jnp.dot(h, w2_ref[...], preferred_element_type=jnp.float32,
                         precision=lax.Precision.DEFAULT)


def _tc_mm(agg, norm3, W1, b1, W2):
  return pl.pallas_call(
      _mm_body,
      grid=(N // _MM_BLK,),
      in_specs=[
          pl.BlockSpec((NC, _MM_BLK, HALF), lambda i: (0, i, 0)),
          pl.BlockSpec((1, _MM_BLK, 1), lambda i: (1, i, 0)),
          pl.BlockSpec((F_IN, F_HID), lambda i: (0, 0)),
          pl.BlockSpec((F_HID,), lambda i: (0,)),
          pl.BlockSpec((F_HID, F_OUT), lambda i: (0, 0)),
      ],
      out_specs=pl.BlockSpec((_MM_BLK, F_OUT), lambda i: (i, 0)),
      out_shape=jax.ShapeDtypeStruct((N, F_OUT), jnp.float32),
  )(agg, norm3, W1, b1, W2)


def _out_body(agg_ref, ndst_ref, b2_ref, out_ref):
  o = jnp.concatenate([agg_ref[0], agg_ref[1]], axis=-1)
  out_ref[...] = o * ndst_ref[0] + b2_ref[...][None, :]


def _tc_out(agg, norm3, b2):
  return pl.pallas_call(
      _out_body,
      grid=(N // _MM_BLK,),
      in_specs=[
          pl.BlockSpec((NC, _MM_BLK, HALF), lambda i: (0, i, 0)),
          pl.BlockSpec((1, _MM_BLK, 1), lambda i: (1, i, 0)),
          pl.BlockSpec((F_OUT,), lambda i: (0,)),
      ],
      out_specs=pl.BlockSpec((_MM_BLK, F_OUT), lambda i: (i, 0)),
      out_shape=jax.ShapeDtypeStruct((N, F_OUT), jnp.float32),
  )(agg, norm3, b2)


def kernel(node_feats, edge_index, edge_weight, W1, b1, W2, b2):
  ei = edge_index.astype(jnp.int32)
  src = ei[0]
  dst = ei[1]
  w = edge_weight.astype(jnp.float32)

  cnt = _sc_degrees(src, dst)
  norm3 = _tc_norm(cnt)
  nsrc = norm3[0, :, 0]

  agg1 = _sc_agg(node_feats.reshape(NC * N, HALF), src, dst, w, nsrc)
  g2 = _tc_mm(agg1, norm3, W1, b1, W2)
  agg2 = _sc_agg(g2.reshape(NC * N, HALF), src, dst, w, nsrc)
  return _tc_out(agg2, norm3, b2)
